# Initial kernel scaffold; baseline (speedup 1.0000x reference)
import functools

import jax
import jax.numpy as jnp
from jax import lax
from jax.experimental import pallas as pl
from jax.experimental.pallas import tpu as pltpu
from jax.experimental.pallas import tpu_sc as plsc

VOCAB = 30522
HID = 768
MAX_POS = 512
BATCH = 128
SEQ = 512
EPS = 1e-12

NC, NS = 2, 16
NW = NC * NS
N_TOK = BATCH * SEQ
TPW = N_TOK // NW
KCH = 64
NCHUNK = TPW // KCH


def _sc_gather(table, idx_flat):
    mesh = plsc.VectorSubcoreMesh(core_axis_name="c", subcore_axis_name="s")

    @functools.partial(
        pl.kernel,
        out_type=jax.ShapeDtypeStruct((N_TOK, HID), jnp.float32),
        mesh=mesh,
        scratch_types=[
            pltpu.VMEM((TPW,), jnp.int32),
            pltpu.VMEM((2, KCH, HID), jnp.float32),
            pltpu.SemaphoreType.DMA,
            pltpu.SemaphoreType.DMA,
        ],
    )
    def k(table_hbm, idx_hbm, out_hbm, idx_v, rows_v, gsem, wsem):
        wid = lax.axis_index("s") * NC + lax.axis_index("c")
        base = wid * TPW
        pltpu.sync_copy(idx_hbm.at[pl.ds(base, TPW)], idx_v)

        def start_gather(c):
            pltpu.make_async_copy(
                table_hbm.at[idx_v.at[pl.ds(c * KCH, KCH)]],
                rows_v.at[c % 2],
                gsem,
            ).start()

        def wait_gather(c):
            pltpu.make_async_copy(
                table_hbm.at[idx_v.at[pl.ds(c * KCH, KCH)]],
                rows_v.at[c % 2],
                gsem,
            ).wait()

        def start_write(c):
            pltpu.make_async_copy(
                rows_v.at[c % 2],
                out_hbm.at[pl.ds(base + c * KCH, KCH)],
                wsem,
            ).start()

        def wait_write(c):
            pltpu.make_async_copy(
                rows_v.at[c % 2],
                out_hbm.at[pl.ds(base + c * KCH, KCH)],
                wsem,
            ).wait()

        start_gather(0)
        for c in range(NCHUNK):
            wait_gather(c)
            if c + 1 < NCHUNK:
                if c >= 1:
                    wait_write(c - 1)
                start_gather(c + 1)
            start_write(c)
        wait_write(NCHUNK - 2)
        wait_write(NCHUNK - 1)

    return k(table, idx_flat)


BB = 8
BS = 64


def _tc_add_ln(gathered, segment_ids, segment_table, position_table, gamma, beta):
    def body(g_ref, sid_ref, seg_ref, pos_ref, gam_ref, bet_ref, out_ref):
        x = g_ref[...]
        sid = sid_ref[...]
        seg = seg_ref[...]
        pos = pos_ref[...]
        e = x + pos[None, :, :] + jnp.where(
            (sid == 0)[:, :, None], seg[0][None, None, :], seg[1][None, None, :]
        )
        mu = jnp.mean(e, axis=-1, keepdims=True)
        var = jnp.mean((e - mu) ** 2, axis=-1, keepdims=True)
        o = (e - mu) * lax.rsqrt(var + EPS)
        out_ref[...] = o * gam_ref[0][None, None, :] + bet_ref[0][None, None, :]

    grid = (BATCH // BB, SEQ // BS)
    return pl.pallas_call(
        body,
        grid=grid,
        in_specs=[
            pl.BlockSpec((BB, BS, HID), lambda i, j: (i, j, 0)),
            pl.BlockSpec((BB, BS), lambda i, j: (i, j)),
            pl.BlockSpec((2, HID), lambda i, j: (0, 0)),
            pl.BlockSpec((BS, HID), lambda i, j: (j, 0)),
            pl.BlockSpec((1, HID), lambda i, j: (0, 0)),
            pl.BlockSpec((1, HID), lambda i, j: (0, 0)),
        ],
        out_specs=pl.BlockSpec((BB, BS, HID), lambda i, j: (i, j, 0)),
        out_shape=jax.ShapeDtypeStruct((BATCH, SEQ, HID), jnp.float32),
    )(gathered, segment_ids, segment_table, position_table, gamma, beta)


def kernel(input_ids, segment_ids, token_table, segment_table, position_table, gamma, beta):
    idx_flat = input_ids.astype(jnp.int32).reshape(-1)
    gathered = _sc_gather(token_table, idx_flat)
    gathered = gathered.reshape(BATCH, SEQ, HID)
    out = _tc_add_ln(
        gathered,
        segment_ids.astype(jnp.int32),
        segment_table,
        position_table,
        gamma.reshape(1, HID),
        beta.reshape(1, HID),
    )
    return out


# speedup vs baseline: 2.1861x; 2.1861x over previous
import functools

import jax
import jax.numpy as jnp
from jax import lax
from jax.experimental import pallas as pl
from jax.experimental.pallas import tpu as pltpu
from jax.experimental.pallas import tpu_sc as plsc

VOCAB = 30522
HID = 768
MAX_POS = 512
BATCH = 128
SEQ = 512
EPS = 1e-12

NC, NS = 2, 16
NW = NC * NS
N_TOK = BATCH * SEQ
TPW = N_TOK // NW
KCH = 64
NCHUNK = TPW // KCH


def _sc_gather(table, idx_flat):
    mesh = plsc.VectorSubcoreMesh(core_axis_name="c", subcore_axis_name="s")

    @functools.partial(
        pl.kernel,
        out_type=jax.ShapeDtypeStruct((N_TOK, HID), jnp.float32),
        mesh=mesh,
        scratch_types=[
            pltpu.VMEM((TPW,), jnp.int32),
            pltpu.VMEM((2, KCH, HID), jnp.float32),
            pltpu.SemaphoreType.DMA,
            pltpu.SemaphoreType.DMA,
        ],
    )
    def k(table_hbm, idx_hbm, out_hbm, idx_v, rows_v, gsem, wsem):
        wid = lax.axis_index("s") * NC + lax.axis_index("c")
        base = wid * TPW
        pltpu.sync_copy(idx_hbm.at[pl.ds(base, TPW)], idx_v)

        def start_gather(c):
            pltpu.make_async_copy(
                table_hbm.at[idx_v.at[pl.ds(c * KCH, KCH)]],
                rows_v.at[c % 2],
                gsem,
            ).start()

        def wait_gather(c):
            pltpu.make_async_copy(
                table_hbm.at[idx_v.at[pl.ds(c * KCH, KCH)]],
                rows_v.at[c % 2],
                gsem,
            ).wait()

        def start_write(c):
            pltpu.make_async_copy(
                rows_v.at[c % 2],
                out_hbm.at[pl.ds(base + c * KCH, KCH)],
                wsem,
            ).start()

        def wait_write(c):
            pltpu.make_async_copy(
                rows_v.at[c % 2],
                out_hbm.at[pl.ds(base + c * KCH, KCH)],
                wsem,
            ).wait()

        start_gather(0)
        for c in range(NCHUNK):
            wait_gather(c)
            if c + 1 < NCHUNK:
                if c >= 1:
                    wait_write(c - 1)
                start_gather(c + 1)
            start_write(c)
        wait_write(NCHUNK - 2)
        wait_write(NCHUNK - 1)

    return k(table, idx_flat)


BB = 8
BS = 128


def _tc_add_ln(gathered, segment_ids, segment_table, position_table, gamma, beta):
    def body(g_ref, sid_ref, seg_ref, pos_ref, gam_ref, bet_ref, out_ref):
        x = g_ref[...]
        sidf = sid_ref[...].astype(jnp.float32)
        seg = seg_ref[...]
        pos = pos_ref[...]
        e = (x + pos[None, :, :] + seg[0][None, None, :]
             + sidf * (seg[1] - seg[0])[None, None, :])
        mu = jnp.mean(e, axis=-1, keepdims=True)
        var = jnp.mean((e - mu) ** 2, axis=-1, keepdims=True)
        o = (e - mu) * lax.rsqrt(var + EPS)
        out_ref[...] = o * gam_ref[0][None, None, :] + bet_ref[0][None, None, :]

    grid = (BATCH // BB, SEQ // BS)
    return pl.pallas_call(
        body,
        grid=grid,
        in_specs=[
            pl.BlockSpec((BB, BS, HID), lambda i, j: (i, j, 0)),
            pl.BlockSpec((BB, BS, 1), lambda i, j: (i, j, 0)),
            pl.BlockSpec((2, HID), lambda i, j: (0, 0)),
            pl.BlockSpec((BS, HID), lambda i, j: (j, 0)),
            pl.BlockSpec((1, HID), lambda i, j: (0, 0)),
            pl.BlockSpec((1, HID), lambda i, j: (0, 0)),
        ],
        out_specs=pl.BlockSpec((BB, BS, HID), lambda i, j: (i, j, 0)),
        out_shape=jax.ShapeDtypeStruct((BATCH, SEQ, HID), jnp.float32),
    )(gathered, segment_ids, segment_table, position_table, gamma, beta)


def kernel(input_ids, segment_ids, token_table, segment_table, position_table, gamma, beta):
    idx_flat = input_ids.astype(jnp.int32).reshape(-1)
    gathered = _sc_gather(token_table, idx_flat)
    gathered = gathered.reshape(BATCH, SEQ, HID)
    out = _tc_add_ln(
        gathered,
        segment_ids.astype(jnp.int32).reshape(BATCH, SEQ, 1),
        segment_table,
        position_table,
        gamma.reshape(1, HID),
        beta.reshape(1, HID),
    )
    return out


# speedup vs baseline: 2.2672x; 1.0371x over previous
import functools

import jax
import jax.numpy as jnp
from jax import lax
from jax.experimental import pallas as pl
from jax.experimental.pallas import tpu as pltpu
from jax.experimental.pallas import tpu_sc as plsc

VOCAB = 30522
HID = 768
MAX_POS = 512
BATCH = 128
SEQ = 512
EPS = 1e-12

NC, NS = 2, 16
NW = NC * NS
N_TOK = BATCH * SEQ
TPW = N_TOK // NW
KCH = 64
NCHUNK = TPW // KCH


def _sc_gather(table, idx_flat):
    mesh = plsc.VectorSubcoreMesh(core_axis_name="c", subcore_axis_name="s")

    @functools.partial(
        pl.kernel,
        out_type=jax.ShapeDtypeStruct((N_TOK, HID), jnp.float32),
        mesh=mesh,
        scratch_types=[
            pltpu.VMEM((TPW,), jnp.int32),
            pltpu.VMEM((2, KCH, HID), jnp.float32),
            pltpu.SemaphoreType.DMA,
            pltpu.SemaphoreType.DMA,
        ],
    )
    def k(table_hbm, idx_hbm, out_hbm, idx_v, rows_v, gsem, wsem):
        wid = lax.axis_index("s") * NC + lax.axis_index("c")
        base = wid * TPW
        pltpu.sync_copy(idx_hbm.at[pl.ds(base, TPW)], idx_v)

        def start_gather(c):
            pltpu.make_async_copy(
                table_hbm.at[idx_v.at[pl.ds(c * KCH, KCH)]],
                rows_v.at[c % 2],
                gsem,
            ).start()

        def wait_gather(c):
            pltpu.make_async_copy(
                table_hbm.at[idx_v.at[pl.ds(c * KCH, KCH)]],
                rows_v.at[c % 2],
                gsem,
            ).wait()

        def start_write(c):
            pltpu.make_async_copy(
                rows_v.at[c % 2],
                out_hbm.at[pl.ds(base + c * KCH, KCH)],
                wsem,
            ).start()

        def wait_write(c):
            pltpu.make_async_copy(
                rows_v.at[c % 2],
                out_hbm.at[pl.ds(base + c * KCH, KCH)],
                wsem,
            ).wait()

        start_gather(0)
        for c in range(NCHUNK):
            wait_gather(c)
            if c + 1 < NCHUNK:
                if c >= 1:
                    wait_write(c - 1)
                start_gather(c + 1)
            start_write(c)
        wait_write(NCHUNK - 2)
        wait_write(NCHUNK - 1)

    return k(table, idx_flat)


BB = 8
BS = 256


def _tc_add_ln(gathered, segment_ids, segment_table, position_table, gamma, beta):
    def body(g_ref, sid_ref, seg_ref, pos_ref, gam_ref, bet_ref, out_ref):
        x = g_ref[...]
        sidf = sid_ref[...].astype(jnp.float32)
        seg = seg_ref[...]
        pos = pos_ref[...]
        e = (x + pos[None, :, :] + seg[0][None, None, :]
             + sidf * (seg[1] - seg[0])[None, None, :])
        mu = jnp.mean(e, axis=-1, keepdims=True)
        var = jnp.mean((e - mu) ** 2, axis=-1, keepdims=True)
        o = (e - mu) * lax.rsqrt(var + EPS)
        out_ref[...] = o * gam_ref[0][None, None, :] + bet_ref[0][None, None, :]

    grid = (BATCH // BB, SEQ // BS)
    return pl.pallas_call(
        body,
        grid=grid,
        in_specs=[
            pl.BlockSpec((BB, BS, HID), lambda i, j: (i, j, 0)),
            pl.BlockSpec((BB, BS, 1), lambda i, j: (i, j, 0)),
            pl.BlockSpec((2, HID), lambda i, j: (0, 0)),
            pl.BlockSpec((BS, HID), lambda i, j: (j, 0)),
            pl.BlockSpec((1, HID), lambda i, j: (0, 0)),
            pl.BlockSpec((1, HID), lambda i, j: (0, 0)),
        ],
        out_specs=pl.BlockSpec((BB, BS, HID), lambda i, j: (i, j, 0)),
        out_shape=jax.ShapeDtypeStruct((BATCH, SEQ, HID), jnp.float32),
    )(gathered, segment_ids, segment_table, position_table, gamma, beta)


def kernel(input_ids, segment_ids, token_table, segment_table, position_table, gamma, beta):
    idx_flat = input_ids.astype(jnp.int32).reshape(-1)
    gathered = _sc_gather(token_table, idx_flat)
    gathered = gathered.reshape(BATCH, SEQ, HID)
    out = _tc_add_ln(
        gathered,
        segment_ids.astype(jnp.int32).reshape(BATCH, SEQ, 1),
        segment_table,
        position_table,
        gamma.reshape(1, HID),
        beta.reshape(1, HID),
    )
    return out


# speedup vs baseline: 2.2738x; 1.0029x over previous
import functools

import jax
import jax.numpy as jnp
from jax import lax
from jax.experimental import pallas as pl
from jax.experimental.pallas import tpu as pltpu
from jax.experimental.pallas import tpu_sc as plsc

VOCAB = 30522
HID = 768
MAX_POS = 512
BATCH = 128
SEQ = 512
EPS = 1e-12

NC, NS = 2, 16
NW = NC * NS
N_TOK = BATCH * SEQ
KCH = 64
NSPLIT = 2
BSPLIT = BATCH // NSPLIT


def _sc_gather(table, idx_flat):
    n_tok = idx_flat.shape[0]
    tpw = n_tok // NW
    nchunk = tpw // KCH
    mesh = plsc.VectorSubcoreMesh(core_axis_name="c", subcore_axis_name="s")

    @functools.partial(
        pl.kernel,
        out_type=jax.ShapeDtypeStruct((n_tok, HID), jnp.float32),
        mesh=mesh,
        scratch_types=[
            pltpu.VMEM((tpw,), jnp.int32),
            pltpu.VMEM((2, KCH, HID), jnp.float32),
            pltpu.SemaphoreType.DMA,
            pltpu.SemaphoreType.DMA,
        ],
    )
    def k(table_hbm, idx_hbm, out_hbm, idx_v, rows_v, gsem, wsem):
        wid = lax.axis_index("s") * NC + lax.axis_index("c")
        base = wid * tpw
        pltpu.sync_copy(idx_hbm.at[pl.ds(base, tpw)], idx_v)

        def start_gather(c):
            pltpu.make_async_copy(
                table_hbm.at[idx_v.at[pl.ds(c * KCH, KCH)]],
                rows_v.at[c % 2],
                gsem,
            ).start()

        def wait_gather(c):
            pltpu.make_async_copy(
                table_hbm.at[idx_v.at[pl.ds(c * KCH, KCH)]],
                rows_v.at[c % 2],
                gsem,
            ).wait()

        def start_write(c):
            pltpu.make_async_copy(
                rows_v.at[c % 2],
                out_hbm.at[pl.ds(base + c * KCH, KCH)],
                wsem,
            ).start()

        def wait_write(c):
            pltpu.make_async_copy(
                rows_v.at[c % 2],
                out_hbm.at[pl.ds(base + c * KCH, KCH)],
                wsem,
            ).wait()

        start_gather(0)
        for c in range(nchunk):
            wait_gather(c)
            if c + 1 < nchunk:
                if c >= 1:
                    wait_write(c - 1)
                start_gather(c + 1)
            start_write(c)
        wait_write(nchunk - 2)
        wait_write(nchunk - 1)

    return k(table, idx_flat)


BB = 8
BS = 256


def _tc_add_ln(gathered, segment_ids, segment_table, position_table, gamma, beta,
               b_off, acc):
    def body(g_ref, sid_ref, seg_ref, pos_ref, gam_ref, bet_ref, *rest):
        out_ref = rest[-1]
        x = g_ref[...]
        sidf = sid_ref[...].astype(jnp.float32)
        seg = seg_ref[...]
        pos = pos_ref[...]
        e = (x + pos[None, :, :] + seg[0][None, None, :]
             + sidf * (seg[1] - seg[0])[None, None, :])
        mu = jnp.mean(e, axis=-1, keepdims=True)
        var = jnp.mean((e - mu) ** 2, axis=-1, keepdims=True)
        o = (e - mu) * lax.rsqrt(var + EPS)
        out_ref[...] = o * gam_ref[0][None, None, :] + bet_ref[0][None, None, :]

    grid = (BSPLIT // BB, SEQ // BS)
    ob = b_off // BB
    in_specs = [
        pl.BlockSpec((BB, BS, HID), lambda i, j: (i, j, 0)),
        pl.BlockSpec((BB, BS, 1), lambda i, j: (i, j, 0)),
        pl.BlockSpec((2, HID), lambda i, j: (0, 0)),
        pl.BlockSpec((BS, HID), lambda i, j: (j, 0)),
        pl.BlockSpec((1, HID), lambda i, j: (0, 0)),
        pl.BlockSpec((1, HID), lambda i, j: (0, 0)),
    ]
    args = [gathered, segment_ids, segment_table, position_table, gamma, beta]
    aliases = {}
    if acc is not None:
        in_specs.append(pl.BlockSpec(memory_space=pl.ANY))
        args.append(acc)
        aliases = {6: 0}
    return pl.pallas_call(
        body,
        grid=grid,
        in_specs=in_specs,
        out_specs=pl.BlockSpec((BB, BS, HID), lambda i, j: (i + ob, j, 0)),
        out_shape=jax.ShapeDtypeStruct((BATCH, SEQ, HID), jnp.float32),
        input_output_aliases=aliases,
    )(*args)


def kernel(input_ids, segment_ids, token_table, segment_table, position_table, gamma, beta):
    ids = input_ids.astype(jnp.int32)
    sids = segment_ids.astype(jnp.int32).reshape(BATCH, SEQ, 1)
    gamma2 = gamma.reshape(1, HID)
    beta2 = beta.reshape(1, HID)
    out = None
    for s in range(NSPLIT):
        b0 = s * BSPLIT
        g = _sc_gather(token_table, ids[b0:b0 + BSPLIT].reshape(-1))
        out = _tc_add_ln(
            g.reshape(BSPLIT, SEQ, HID),
            sids[b0:b0 + BSPLIT],
            segment_table,
            position_table,
            gamma2,
            beta2,
            b0,
            out,
        )
    return out
